# baseline (speedup 1.0000x reference)
import jax
import jax.numpy as jnp
from jax import lax
from jax.experimental import pallas as pl
from jax.experimental.pallas import tpu as pltpu, tpu_sc as plsc

B = 16384
D = 64
NEG = 10
W = 2 * D
NW = 32
CHUNK = B // NW
SB = 64
NCH = CHUNK // SB
VL = 16


def _sc_body(in_hbm, out_hbm, c_hbm, p_hbm, n_hbm,
             ps_hbm, ns_hbm,
             craw, praw, nraw, cidx, pidx, nidx,
             vrows, prows, nrows, psc, nsc, sem):
    wid = lax.axis_index("s") * 2 + lax.axis_index("c")
    wbase = wid * CHUNK
    vh_in = in_hbm.shape[0]
    vh_out = out_hbm.shape[0]

    @pl.loop(0, NCH)
    def _(c):
        base = wbase + c * SB
        pltpu.sync_copy(c_hbm.at[pl.ds(base, SB)], craw)
        pltpu.sync_copy(p_hbm.at[pl.ds(base, SB)], praw)
        for j in range(NEG):
            pltpu.sync_copy(n_hbm.at[pl.ds(j * B + base, SB)], nraw.at[j])
        for k in range(SB // VL):
            sl = pl.ds(k * VL, VL)
            v = craw[sl]
            cidx[sl] = v - (1 + ((v - vh_in) >> 31)) * vh_in
            v = praw[sl]
            pidx[sl] = v - (1 + ((v - vh_out) >> 31)) * vh_out
            for j in range(NEG):
                v = nraw[j, sl]
                nidx[j, sl] = v - (1 + ((v - vh_out) >> 31)) * vh_out
        descs = [pltpu.async_copy(in_hbm.at[cidx], vrows, sem),
                 pltpu.async_copy(out_hbm.at[pidx], prows, sem)]
        for j in range(NEG):
            descs.append(
                pltpu.async_copy(out_hbm.at[nidx.at[j]], nrows.at[j], sem))
        for dsc in descs:
            dsc.wait()

        lane = lax.iota(jnp.int32, VL)

        dnums = lax.GatherDimensionNumbers(
            offset_dims=(), collapsed_slice_dims=(0,), start_index_map=(0,))

        def perm(x, idx):
            return lax.gather(x, idx[:, None], dnums, (1,),
                              unique_indices=True, indices_are_sorted=False,
                              mode=lax.GatherScatterMode.PROMISE_IN_BOUNDS)

        def hsum(x):
            for sh in (8, 4, 2, 1):
                x = x + perm(x, lane ^ sh)
            return x

        @pl.loop(0, SB // VL)
        def _(g):
            gb = g * VL
            pcv = 1 + ((craw[pl.ds(gb, VL)] - vh_in) >> 31)
            ppv = 1 + ((praw[pl.ds(gb, VL)] - vh_out) >> 31)
            npv = [1 + ((nraw[j, pl.ds(gb, VL)] - vh_out) >> 31)
                   for j in range(NEG)]
            ps_acc = jnp.zeros(VL, jnp.float32)
            ns_acc = [jnp.zeros(VL, jnp.float32) for _ in range(NEG)]
            for i in range(VL):
                e = gb + i
                coff = pcv[i] * D
                v = [vrows[e, pl.ds(coff + k * VL, VL)]
                     for k in range(D // VL)]
                poff = ppv[i] * D
                t = v[0] * prows[e, pl.ds(poff, VL)]
                for k in range(1, D // VL):
                    t = t + v[k] * prows[e, pl.ds(poff + k * VL, VL)]
                ps_acc = jnp.where(lane == i, hsum(t), ps_acc)
                for j in range(NEG):
                    noff = npv[j][i] * D
                    t = v[0] * nrows[j, e, pl.ds(noff, VL)]
                    for k in range(1, D // VL):
                        t = t + v[k] * nrows[j, e, pl.ds(noff + k * VL, VL)]
                    ns_acc[j] = jnp.where(lane == i, hsum(t), ns_acc[j])
            psc[pl.ds(gb, VL)] = ps_acc
            for j in range(NEG):
                nsc[j, pl.ds(gb, VL)] = ns_acc[j]

        pltpu.sync_copy(psc, ps_hbm.at[pl.ds(base, SB)])
        for j in range(NEG):
            pltpu.sync_copy(nsc.at[j], ns_hbm.at[pl.ds(j * B + base, SB)])


def _sc_scores(in2, out2, c, p, n):
    mesh = plsc.VectorSubcoreMesh(core_axis_name="c", subcore_axis_name="s")
    fn = pl.kernel(
        _sc_body,
        out_type=(jax.ShapeDtypeStruct((B,), jnp.float32),
                  jax.ShapeDtypeStruct((NEG * B,), jnp.float32)),
        mesh=mesh,
        scratch_types=[
            pltpu.VMEM((SB,), jnp.int32),
            pltpu.VMEM((SB,), jnp.int32),
            pltpu.VMEM((NEG, SB), jnp.int32),
            pltpu.VMEM((SB,), jnp.int32),
            pltpu.VMEM((SB,), jnp.int32),
            pltpu.VMEM((NEG, SB), jnp.int32),
            pltpu.VMEM((SB, W), jnp.float32),
            pltpu.VMEM((SB, W), jnp.float32),
            pltpu.VMEM((NEG, SB, W), jnp.float32),
            pltpu.VMEM((SB,), jnp.float32),
            pltpu.VMEM((NEG, SB), jnp.float32),
            pltpu.SemaphoreType.DMA,
        ],
    )
    return fn(in2, out2, c, p, n)


def _repack_body(a_ref, b_ref, o_ref):
    o_ref[...] = jnp.concatenate([a_ref[...], b_ref[...]], axis=1)


def _repack(t):
    n = t.shape[0] // 2
    r = 5000
    g = n // r
    return pl.pallas_call(
        _repack_body,
        grid=(g,),
        in_specs=[pl.BlockSpec((r, D), lambda i: (i, 0)),
                  pl.BlockSpec((r, D), lambda i: (i + g, 0))],
        out_specs=pl.BlockSpec((r, W), lambda i: (i, 0)),
        out_shape=jax.ShapeDtypeStruct((n, W), jnp.float32),
    )(t, t)


def _loss_body(ps_ref, ns_ref, o_ref):
    p = ps_ref[...]
    n = ns_ref[...]
    ls_p = jnp.minimum(p, 0.0) - jnp.log1p(jnp.exp(-jnp.abs(p)))
    ls_n = jnp.minimum(-n, 0.0) - jnp.log1p(jnp.exp(-jnp.abs(n)))
    o_ref[0, 0] = -(jnp.sum(ls_p) / B) - (jnp.sum(ls_n) / (B * NEG))


def _tc_loss(ps, ns):
    return pl.pallas_call(
        _loss_body,
        in_specs=[pl.BlockSpec((B // 128, 128), lambda: (0, 0)),
                  pl.BlockSpec((NEG * B // 128, 128), lambda: (0, 0))],
        out_specs=pl.BlockSpec(memory_space=pltpu.SMEM),
        out_shape=jax.ShapeDtypeStruct((1, 1), jnp.float32),
    )(ps.reshape(B // 128, 128), ns.reshape(NEG * B // 128, 128))


def kernel(in_embed, out_embed, center, pos, neg):
    center = center.astype(jnp.int32)
    pos = pos.astype(jnp.int32)
    neg_t = neg.astype(jnp.int32).T.reshape(-1)
    in2 = _repack(in_embed)
    out2 = _repack(out_embed)
    ps, ns = _sc_scores(in2, out2, center, pos, neg_t)
    return _tc_loss(ps, ns)[0, 0]


# speedup vs baseline: 1.0609x; 1.0609x over previous
import jax
import jax.numpy as jnp
from jax import lax
from jax.experimental import pallas as pl
from jax.experimental.pallas import tpu as pltpu, tpu_sc as plsc

B = 16384
D = 64
NEG = 10
W = 2 * D
NW = 32
CHUNK = B // NW
SB = 64
NCH = CHUNK // SB
VL = 16


def _sc_body(in_hbm, out_hbm, c_hbm, p_hbm, n_hbm,
             ps_hbm, ns_hbm,
             craw, praw, nraw, cidx, pidx, nidx,
             vrows, prows, nrows, psc, nsc, sem):
    wid = lax.axis_index("s") * 2 + lax.axis_index("c")
    wbase = wid * CHUNK

    @pl.loop(0, NCH)
    def _(c):
        base = wbase + c * SB
        pltpu.sync_copy(c_hbm.at[pl.ds(base, SB)], craw)
        pltpu.sync_copy(p_hbm.at[pl.ds(base, SB)], praw)
        for j in range(NEG):
            pltpu.sync_copy(n_hbm.at[pl.ds(j * B + base, SB)], nraw.at[j])
        for k in range(SB // VL):
            sl = pl.ds(k * VL, VL)
            cidx[sl] = craw[sl] >> 1
            pidx[sl] = praw[sl] >> 1
            for j in range(NEG):
                nidx[j, sl] = nraw[j, sl] >> 1
        descs = [pltpu.async_copy(in_hbm.at[cidx], vrows, sem),
                 pltpu.async_copy(out_hbm.at[pidx], prows, sem)]
        for j in range(NEG):
            descs.append(
                pltpu.async_copy(out_hbm.at[nidx.at[j]], nrows.at[j], sem))
        for dsc in descs:
            dsc.wait()

        lane = lax.iota(jnp.int32, VL)

        dnums = lax.GatherDimensionNumbers(
            offset_dims=(), collapsed_slice_dims=(0,), start_index_map=(0,))

        def perm(x, idx):
            return lax.gather(x, idx[:, None], dnums, (1,),
                              unique_indices=True, indices_are_sorted=False,
                              mode=lax.GatherScatterMode.PROMISE_IN_BOUNDS)

        def hsum(x):
            for sh in (8, 4, 2, 1):
                x = x + perm(x, lane ^ sh)
            return x

        @pl.loop(0, SB // VL)
        def _(g):
            gb = g * VL
            pcv = craw[pl.ds(gb, VL)] & 1
            ppv = praw[pl.ds(gb, VL)] & 1
            npv = [nraw[j, pl.ds(gb, VL)] & 1 for j in range(NEG)]
            ps_acc = jnp.zeros(VL, jnp.float32)
            ns_acc = [jnp.zeros(VL, jnp.float32) for _ in range(NEG)]
            for i in range(VL):
                e = gb + i
                coff = pcv[i] * D
                v = [vrows[e, pl.ds(coff + k * VL, VL)]
                     for k in range(D // VL)]
                poff = ppv[i] * D
                t = v[0] * prows[e, pl.ds(poff, VL)]
                for k in range(1, D // VL):
                    t = t + v[k] * prows[e, pl.ds(poff + k * VL, VL)]
                ps_acc = jnp.where(lane == i, hsum(t), ps_acc)
                for j in range(NEG):
                    noff = npv[j][i] * D
                    t = v[0] * nrows[j, e, pl.ds(noff, VL)]
                    for k in range(1, D // VL):
                        t = t + v[k] * nrows[j, e, pl.ds(noff + k * VL, VL)]
                    ns_acc[j] = jnp.where(lane == i, hsum(t), ns_acc[j])
            psc[pl.ds(gb, VL)] = ps_acc
            for j in range(NEG):
                nsc[j, pl.ds(gb, VL)] = ns_acc[j]

        pltpu.sync_copy(psc, ps_hbm.at[pl.ds(base, SB)])
        for j in range(NEG):
            pltpu.sync_copy(nsc.at[j], ns_hbm.at[pl.ds(j * B + base, SB)])


def _sc_scores(in2, out2, c, p, n):
    mesh = plsc.VectorSubcoreMesh(core_axis_name="c", subcore_axis_name="s")
    fn = pl.kernel(
        _sc_body,
        out_type=(jax.ShapeDtypeStruct((B,), jnp.float32),
                  jax.ShapeDtypeStruct((NEG * B,), jnp.float32)),
        mesh=mesh,
        scratch_types=[
            pltpu.VMEM((SB,), jnp.int32),
            pltpu.VMEM((SB,), jnp.int32),
            pltpu.VMEM((NEG, SB), jnp.int32),
            pltpu.VMEM((SB,), jnp.int32),
            pltpu.VMEM((SB,), jnp.int32),
            pltpu.VMEM((NEG, SB), jnp.int32),
            pltpu.VMEM((SB, W), jnp.float32),
            pltpu.VMEM((SB, W), jnp.float32),
            pltpu.VMEM((NEG, SB, W), jnp.float32),
            pltpu.VMEM((SB,), jnp.float32),
            pltpu.VMEM((NEG, SB), jnp.float32),
            pltpu.SemaphoreType.DMA,
        ],
    )
    return fn(in2, out2, c, p, n)


def _loss_body(ps_ref, ns_ref, o_ref):
    p = ps_ref[...]
    n = ns_ref[...]
    ls_p = jnp.minimum(p, 0.0) - jnp.log1p(jnp.exp(-jnp.abs(p)))
    ls_n = jnp.minimum(-n, 0.0) - jnp.log1p(jnp.exp(-jnp.abs(n)))
    o_ref[0, 0] = -(jnp.sum(ls_p) / B) - (jnp.sum(ls_n) / (B * NEG))


def _tc_loss(ps, ns):
    return pl.pallas_call(
        _loss_body,
        in_specs=[pl.BlockSpec((B // 128, 128), lambda: (0, 0)),
                  pl.BlockSpec((NEG * B // 128, 128), lambda: (0, 0))],
        out_specs=pl.BlockSpec(memory_space=pltpu.SMEM),
        out_shape=jax.ShapeDtypeStruct((1, 1), jnp.float32),
    )(ps.reshape(B // 128, 128), ns.reshape(NEG * B // 128, 128))


def kernel(in_embed, out_embed, center, pos, neg):
    center = center.astype(jnp.int32)
    pos = pos.astype(jnp.int32)
    neg_t = neg.astype(jnp.int32).T.reshape(-1)
    in2 = in_embed.reshape(-1, W)
    out2 = out_embed.reshape(-1, W)
    ps, ns = _sc_scores(in2, out2, center, pos, neg_t)
    return _tc_loss(ps, ns)[0, 0]
